# Initial kernel scaffold; baseline (speedup 1.0000x reference)
import jax
import jax.numpy as jnp
from jax.experimental import pallas as pl


def kernel(x, edge_index, edge_attr, Wp, bp, Wln, bln, We1, be1, We2, be2, Wih, Whh, bih, bhh):
    raise NotImplementedError("write your pallas kernel here")


# speedup vs baseline: 3.3758x; 3.3758x over previous
import functools

import jax
import jax.numpy as jnp
import numpy as np
from jax import lax
from jax.experimental import pallas as pl
from jax.experimental.pallas import tpu as pltpu
from jax.experimental.pallas import tpu_sc as plsc

N = 10000
E = 160000
H = 16
EH = 128
DIN = 128
STEPS = 3

NC, NS = 2, 16
NW = NC * NS
EPW = E // NW
CHUNK = 125
NCHUNK = EPW // CHUNK
NPAD = 10240
RPT = NPAD // NS

@functools.cache
def _sc_kernels():
    mesh = plsc.VectorSubcoreMesh(core_axis_name="c", subcore_axis_name="s",
                                  num_cores=NC, num_subcores=NS)
    params = pltpu.CompilerParams(use_tc_tiling_on_sc=False)

    @functools.partial(
        pl.kernel,
        out_type=jax.ShapeDtypeStruct((E, H), jnp.float32),
        mesh=mesh,
        scratch_types=[
            pltpu.VMEM((EPW,), jnp.int32),
            pltpu.VMEM((EPW, H), jnp.float32),
            pltpu.SemaphoreType.DMA,
        ],
        compiler_params=params,
    )
    def sc_gather(h_hbm, src_hbm, out_hbm, idx_v, rows_v, sem):
        wid = lax.axis_index("s") * NC + lax.axis_index("c")
        base = wid * EPW
        pltpu.sync_copy(src_hbm.at[pl.ds(base, EPW)], idx_v)
        pltpu.async_copy(h_hbm.at[idx_v], rows_v, sem).wait()
        pltpu.sync_copy(rows_v, out_hbm.at[pl.ds(base, EPW)])

    @functools.partial(
        pl.kernel,
        out_type=jax.ShapeDtypeStruct((NC, NPAD, H), jnp.float32),
        mesh=mesh,
        scratch_types=[
            pltpu.VMEM((NCHUNK, CHUNK), jnp.int32),
            pltpu.VMEM((EPW, H), jnp.float32),
            pltpu.VMEM_SHARED((NPAD, H), jnp.float32),
        ],
        compiler_params=params,
    )
    def sc_scatter(vals_hbm, dst2d_hbm, zeros_hbm, out_hbm, idx_v, vals_v,
                   acc_sh):
        c = lax.axis_index("c")
        s = lax.axis_index("s")
        wid = s * NC + c
        pltpu.sync_copy(zeros_hbm.at[pl.ds(s * RPT, RPT)],
                        acc_sh.at[pl.ds(s * RPT, RPT)])
        pltpu.sync_copy(vals_hbm.at[pl.ds(wid * EPW, EPW)], vals_v)
        pltpu.sync_copy(dst2d_hbm.at[pl.ds(wid * NCHUNK, NCHUNK)], idx_v)
        plsc.subcore_barrier()

        def body(j, carry):
            pltpu.sync_copy(vals_v.at[pl.ds(j * CHUNK, CHUNK)],
                            acc_sh.at[idx_v.at[j]], add=True)
            return carry

        lax.fori_loop(0, NCHUNK, body, 0)
        plsc.subcore_barrier()
        pltpu.sync_copy(acc_sh.at[pl.ds(s * RPT, RPT)],
                        out_hbm.at[c, pl.ds(s * RPT, RPT)])

    return sc_gather, sc_scatter


def _edge_body(ea_ref, xj_ref, w1_ref, b1_ref, w2_ref, b2_ref, br_ref, ss_ref,
               m_ref):
    eh = jnp.dot(ea_ref[...], w1_ref[...], preferred_element_type=jnp.float32)
    eh = jnp.maximum(eh + b1_ref[...], 0.0)
    ew = jnp.dot(eh, w2_ref[...], preferred_element_type=jnp.float32)
    ew = ew + b2_ref[...]
    xjb = jnp.dot(xj_ref[...], br_ref[...], preferred_element_type=jnp.float32)
    m_ref[...] = jnp.dot(xjb * ew, ss_ref[...],
                         preferred_element_type=jnp.float32)


EB = 2000


def _edge_call(ea, xj, w1t, b1, w2t, b2, brep, ssum):
    full = lambda r, c: pl.BlockSpec((r, c), lambda i: (0, 0))
    return pl.pallas_call(
        _edge_body,
        grid=(E // EB,),
        in_specs=[
            pl.BlockSpec((EB, H), lambda i: (i, 0)),
            pl.BlockSpec((EB, H), lambda i: (i, 0)),
            full(H, EH), full(1, EH), full(EH, H * H), full(1, H * H),
            full(H, H * H), full(H * H, H),
        ],
        out_specs=pl.BlockSpec((EB, H), lambda i: (i, 0)),
        out_shape=jax.ShapeDtypeStruct((E, H), jnp.float32),
    )(ea, xj, w1t, b1, w2t, b2, brep, ssum)


def _pro_body(x_ref, wpt_ref, bp_ref, wlnt_ref, bln_ref,
              whr_ref, whz_ref, whn_ref, bhr_ref, bhz_ref, bhn_ref,
              degp_ref,
              nf0_ref, h0_ref, ghr_ref, ghz_ref, ghn_ref, dinv_ref):
    nf0 = jnp.dot(x_ref[...], wpt_ref[...], preferred_element_type=jnp.float32)
    nf0 = jnp.maximum(nf0 + bp_ref[...], 0.0)
    nf0_ref[...] = nf0
    h0_ref[...] = jnp.dot(nf0, wlnt_ref[...],
                          preferred_element_type=jnp.float32) + bln_ref[...]
    ghr_ref[...] = jnp.dot(nf0, whr_ref[...],
                           preferred_element_type=jnp.float32) + bhr_ref[...]
    ghz_ref[...] = jnp.dot(nf0, whz_ref[...],
                           preferred_element_type=jnp.float32) + bhz_ref[...]
    ghn_ref[...] = jnp.dot(nf0, whn_ref[...],
                           preferred_element_type=jnp.float32) + bhn_ref[...]
    deg = degp_ref[0] + degp_ref[1]
    dinv_ref[...] = 1.0 / jnp.maximum(deg, 1.0)


NB = 2000


def _pro_call(x, wpt, bp, wlnt, bln, whr, whz, whn, bhr, bhz, bhn, degp):
    o = jax.ShapeDtypeStruct((N, H), jnp.float32)
    full = lambda r, c: pl.BlockSpec((r, c), lambda i: (0, 0))
    nb = pl.BlockSpec((NB, H), lambda i: (i, 0))
    return pl.pallas_call(
        _pro_body,
        grid=(N // NB,),
        in_specs=[
            pl.BlockSpec((NB, DIN), lambda i: (i, 0)),
            full(DIN, H), full(1, H), full(H, H), full(1, H),
            full(H, H), full(H, H), full(H, H),
            full(1, H), full(1, H), full(1, H),
            pl.BlockSpec((NC, NB, H), lambda i: (0, i, 0)),
        ],
        out_specs=(nb, nb, nb, nb, nb, nb),
        out_shape=(o, o, o, o, o, o),
    )(x, wpt, bp, wlnt, bln, whr, whz, whn, bhr, bhz, bhn, degp)


def _gru_body(p_ref, dinv_ref, ghr_ref, ghz_ref, ghn_ref, hid_ref,
              wir_ref, wiz_ref, win_ref, bir_ref, biz_ref, bin_ref,
              wlnt_ref, bln_ref, nf_ref, h_ref):
    agg = (p_ref[0] + p_ref[1]) * dinv_ref[...]
    nf = jnp.maximum(agg, 0.0)
    r = jax.nn.sigmoid(
        jnp.dot(nf, wir_ref[...], preferred_element_type=jnp.float32)
        + bir_ref[...] + ghr_ref[...])
    z = jax.nn.sigmoid(
        jnp.dot(nf, wiz_ref[...], preferred_element_type=jnp.float32)
        + biz_ref[...] + ghz_ref[...])
    n = jnp.tanh(
        jnp.dot(nf, win_ref[...], preferred_element_type=jnp.float32)
        + bin_ref[...] + r * ghn_ref[...])
    out = (1.0 - z) * n + z * hid_ref[...]
    nf_ref[...] = out
    h_ref[...] = jnp.dot(out, wlnt_ref[...],
                         preferred_element_type=jnp.float32) + bln_ref[...]


def _gru_call(p, dinv, ghr, ghz, ghn, hid, wir, wiz, win, bir, biz, bin_,
              wlnt, bln):
    o = jax.ShapeDtypeStruct((N, H), jnp.float32)
    full = lambda r, c: pl.BlockSpec((r, c), lambda i: (0, 0))
    nb = pl.BlockSpec((NB, H), lambda i: (i, 0))
    return pl.pallas_call(
        _gru_body,
        grid=(N // NB,),
        in_specs=[
            pl.BlockSpec((NC, NB, H), lambda i: (0, i, 0)),
            nb, nb, nb, nb, nb,
            full(H, H), full(H, H), full(H, H),
            full(1, H), full(1, H), full(1, H),
            full(H, H), full(1, H),
        ],
        out_specs=(nb, nb),
        out_shape=(o, o),
    )(p, dinv, ghr, ghz, ghn, hid, wir, wiz, win, bir, biz, bin_, wlnt, bln)


_BREP = np.kron(np.eye(H), np.ones((1, H))).astype(np.float32)
_SSUM = np.kron(np.ones((H, 1)), np.eye(H)).astype(np.float32)


def kernel(x, edge_index, edge_attr, Wp, bp, Wln, bln, We1, be1, We2, be2,
           Wih, Whh, bih, bhh):
    src = edge_index[0]
    dst = edge_index[1]
    dst2d = dst.reshape(NW * NCHUNK, CHUNK)
    ones_rows = jnp.ones((E, H), jnp.float32)
    zeros_rows = jnp.zeros((NPAD, H), jnp.float32)

    brep = jnp.asarray(_BREP)
    ssum = jnp.asarray(_SSUM)
    w1t = We1.T
    b1 = be1.reshape(1, EH)
    w2t = We2.T
    b2 = be2.reshape(1, H * H)
    wpt = Wp.T
    bp2 = bp.reshape(1, H)
    wlnt = Wln.T
    bln2 = bln.reshape(1, H)
    whr, whz, whn = Whh[0:H].T, Whh[H:2 * H].T, Whh[2 * H:3 * H].T
    bhr, bhz, bhn = (bhh[0:H].reshape(1, H), bhh[H:2 * H].reshape(1, H),
                     bhh[2 * H:3 * H].reshape(1, H))
    wir, wiz, win = Wih[0:H].T, Wih[H:2 * H].T, Wih[2 * H:3 * H].T
    bir, biz, bin_ = (bih[0:H].reshape(1, H), bih[H:2 * H].reshape(1, H),
                      bih[2 * H:3 * H].reshape(1, H))

    sc_gather, sc_scatter = _sc_kernels()
    degp = sc_scatter(ones_rows, dst2d, zeros_rows)
    nf, h, ghr, ghz, ghn, dinv = _pro_call(
        x, wpt, bp2, wlnt, bln2, whr, whz, whn, bhr, bhz, bhn, degp)
    hidden = nf
    for _ in range(STEPS):
        xj = sc_gather(h, src)
        m = _edge_call(edge_attr, xj, w1t, b1, w2t, b2, brep, ssum)
        p = sc_scatter(m, dst2d, zeros_rows)
        nf, h = _gru_call(p, dinv, ghr, ghz, ghn, hidden,
                          wir, wiz, win, bir, biz, bin_, wlnt, bln2)
    return nf


# speedup vs baseline: 3.3766x; 1.0002x over previous
import functools

import jax
import jax.numpy as jnp
import numpy as np
from jax import lax
from jax.experimental import pallas as pl
from jax.experimental.pallas import tpu as pltpu
from jax.experimental.pallas import tpu_sc as plsc

N = 10000
E = 160000
H = 16
EH = 128
DIN = 128
STEPS = 3

NC, NS = 2, 16
NW = NC * NS
EPW = E // NW
CHUNK = 125
NCHUNK = EPW // CHUNK
NPAD = 10240
RPT = NPAD // NS

@functools.cache
def _sc_kernels():
    mesh = plsc.VectorSubcoreMesh(core_axis_name="c", subcore_axis_name="s",
                                  num_cores=NC, num_subcores=NS)
    params = pltpu.CompilerParams(use_tc_tiling_on_sc=False)

    @functools.partial(
        pl.kernel,
        out_type=jax.ShapeDtypeStruct((E, H), jnp.float32),
        mesh=mesh,
        scratch_types=[
            pltpu.VMEM((EPW,), jnp.int32),
            pltpu.VMEM((EPW, H), jnp.float32),
            pltpu.SemaphoreType.DMA,
        ],
        compiler_params=params,
    )
    def sc_gather(h_hbm, src_hbm, out_hbm, idx_v, rows_v, sem):
        wid = lax.axis_index("s") * NC + lax.axis_index("c")
        base = wid * EPW
        pltpu.sync_copy(src_hbm.at[pl.ds(base, EPW)], idx_v)
        pltpu.async_copy(h_hbm.at[idx_v], rows_v, sem).wait()
        pltpu.sync_copy(rows_v, out_hbm.at[pl.ds(base, EPW)])

    @functools.partial(
        pl.kernel,
        out_type=jax.ShapeDtypeStruct((NC, NPAD, H), jnp.float32),
        mesh=mesh,
        scratch_types=[
            pltpu.VMEM((NCHUNK, CHUNK), jnp.int32),
            pltpu.VMEM((EPW, H), jnp.float32),
            pltpu.VMEM_SHARED((NPAD, H), jnp.float32),
        ],
        compiler_params=params,
    )
    def sc_scatter(vals_hbm, dst2d_hbm, zeros_hbm, out_hbm, idx_v, vals_v,
                   acc_sh):
        c = lax.axis_index("c")
        s = lax.axis_index("s")
        wid = s * NC + c
        pltpu.sync_copy(zeros_hbm.at[pl.ds(s * RPT, RPT)],
                        acc_sh.at[pl.ds(s * RPT, RPT)])
        pltpu.sync_copy(vals_hbm.at[pl.ds(wid * EPW, EPW)], vals_v)
        pltpu.sync_copy(dst2d_hbm.at[pl.ds(wid * NCHUNK, NCHUNK)], idx_v)
        plsc.subcore_barrier()

        def body(j, carry):
            pltpu.sync_copy(vals_v.at[pl.ds(j * CHUNK, CHUNK)],
                            acc_sh.at[idx_v.at[j]], add=True)
            return carry

        lax.fori_loop(0, NCHUNK, body, 0)
        plsc.subcore_barrier()
        pltpu.sync_copy(acc_sh.at[pl.ds(s * RPT, RPT)],
                        out_hbm.at[c, pl.ds(s * RPT, RPT)])

    return sc_gather, sc_scatter


def _edge_body(ea_ref, xj_ref, w1_ref, b1_ref, w2_ref, b2_ref, br_ref, ss_ref,
               m_ref):
    bf = jnp.bfloat16
    eh = jnp.dot(ea_ref[...].astype(bf), w1_ref[...].astype(bf),
                 preferred_element_type=jnp.float32)
    eh = jnp.maximum(eh + b1_ref[...], 0.0)
    ew = jnp.dot(eh.astype(bf), w2_ref[...].astype(bf),
                 preferred_element_type=jnp.float32)
    ew = ew + b2_ref[...]
    xjb = jnp.dot(xj_ref[...].astype(bf), br_ref[...].astype(bf),
                  preferred_element_type=jnp.float32)
    m_ref[...] = jnp.dot((xjb * ew).astype(bf), ss_ref[...].astype(bf),
                         preferred_element_type=jnp.float32)


EB = 2000


def _edge_call(ea, xj, w1t, b1, w2t, b2, brep, ssum):
    full = lambda r, c: pl.BlockSpec((r, c), lambda i: (0, 0))
    return pl.pallas_call(
        _edge_body,
        grid=(E // EB,),
        in_specs=[
            pl.BlockSpec((EB, H), lambda i: (i, 0)),
            pl.BlockSpec((EB, H), lambda i: (i, 0)),
            full(H, EH), full(1, EH), full(EH, H * H), full(1, H * H),
            full(H, H * H), full(H * H, H),
        ],
        out_specs=pl.BlockSpec((EB, H), lambda i: (i, 0)),
        out_shape=jax.ShapeDtypeStruct((E, H), jnp.float32),
    )(ea, xj, w1t, b1, w2t, b2, brep, ssum)


def _pro_body(x_ref, wpt_ref, bp_ref, wlnt_ref, bln_ref,
              whr_ref, whz_ref, whn_ref, bhr_ref, bhz_ref, bhn_ref,
              nf0_ref, h0_ref, ghr_ref, ghz_ref, ghn_ref):
    nf0 = jnp.dot(x_ref[...], wpt_ref[...], preferred_element_type=jnp.float32)
    nf0 = jnp.maximum(nf0 + bp_ref[...], 0.0)
    nf0_ref[...] = nf0
    h0_ref[...] = jnp.dot(nf0, wlnt_ref[...],
                          preferred_element_type=jnp.float32) + bln_ref[...]
    ghr_ref[...] = jnp.dot(nf0, whr_ref[...],
                           preferred_element_type=jnp.float32) + bhr_ref[...]
    ghz_ref[...] = jnp.dot(nf0, whz_ref[...],
                           preferred_element_type=jnp.float32) + bhz_ref[...]
    ghn_ref[...] = jnp.dot(nf0, whn_ref[...],
                           preferred_element_type=jnp.float32) + bhn_ref[...]


NB = 2000


def _pro_call(x, wpt, bp, wlnt, bln, whr, whz, whn, bhr, bhz, bhn):
    o = jax.ShapeDtypeStruct((N, H), jnp.float32)
    full = lambda r, c: pl.BlockSpec((r, c), lambda i: (0, 0))
    nb = pl.BlockSpec((NB, H), lambda i: (i, 0))
    return pl.pallas_call(
        _pro_body,
        grid=(N // NB,),
        in_specs=[
            pl.BlockSpec((NB, DIN), lambda i: (i, 0)),
            full(DIN, H), full(1, H), full(H, H), full(1, H),
            full(H, H), full(H, H), full(H, H),
            full(1, H), full(1, H), full(1, H),
        ],
        out_specs=(nb, nb, nb, nb, nb),
        out_shape=(o, o, o, o, o),
    )(x, wpt, bp, wlnt, bln, whr, whz, whn, bhr, bhz, bhn)


def _gru_body(p_ref, degp_ref, ghr_ref, ghz_ref, ghn_ref, hid_ref,
              wir_ref, wiz_ref, win_ref, bir_ref, biz_ref, bin_ref,
              wlnt_ref, bln_ref, nf_ref, h_ref):
    deg = jnp.maximum(degp_ref[0] + degp_ref[1], 1.0)
    agg = (p_ref[0] + p_ref[1]) / deg
    nf = jnp.maximum(agg, 0.0)
    r = jax.nn.sigmoid(
        jnp.dot(nf, wir_ref[...], preferred_element_type=jnp.float32)
        + bir_ref[...] + ghr_ref[...])
    z = jax.nn.sigmoid(
        jnp.dot(nf, wiz_ref[...], preferred_element_type=jnp.float32)
        + biz_ref[...] + ghz_ref[...])
    n = jnp.tanh(
        jnp.dot(nf, win_ref[...], preferred_element_type=jnp.float32)
        + bin_ref[...] + r * ghn_ref[...])
    out = (1.0 - z) * n + z * hid_ref[...]
    nf_ref[...] = out
    h_ref[...] = jnp.dot(out, wlnt_ref[...],
                         preferred_element_type=jnp.float32) + bln_ref[...]


def _gru_call(p, degp, ghr, ghz, ghn, hid, wir, wiz, win, bir, biz, bin_,
              wlnt, bln):
    o = jax.ShapeDtypeStruct((N, H), jnp.float32)
    full = lambda r, c: pl.BlockSpec((r, c), lambda i: (0, 0))
    nb = pl.BlockSpec((NB, H), lambda i: (i, 0))
    return pl.pallas_call(
        _gru_body,
        grid=(N // NB,),
        in_specs=[
            pl.BlockSpec((NC, NB, H), lambda i: (0, i, 0)),
            pl.BlockSpec((NC, NB, H), lambda i: (0, i, 0)),
            nb, nb, nb, nb,
            full(H, H), full(H, H), full(H, H),
            full(1, H), full(1, H), full(1, H),
            full(H, H), full(1, H),
        ],
        out_specs=(nb, nb),
        out_shape=(o, o),
    )(p, degp, ghr, ghz, ghn, hid, wir, wiz, win, bir, biz, bin_, wlnt, bln)


_BREP = np.kron(np.eye(H), np.ones((1, H))).astype(np.float32)
_SSUM = np.kron(np.ones((H, 1)), np.eye(H)).astype(np.float32)


def kernel(x, edge_index, edge_attr, Wp, bp, Wln, bln, We1, be1, We2, be2,
           Wih, Whh, bih, bhh):
    src = edge_index[0]
    dst = edge_index[1]
    dst2d = dst.reshape(NW * NCHUNK, CHUNK)
    ones_rows = jnp.ones((E, H), jnp.float32)
    zeros_rows = jnp.zeros((NPAD, H), jnp.float32)

    brep = jnp.asarray(_BREP)
    ssum = jnp.asarray(_SSUM)
    w1t = We1.T
    b1 = be1.reshape(1, EH)
    w2t = We2.T
    b2 = be2.reshape(1, H * H)
    wpt = Wp.T
    bp2 = bp.reshape(1, H)
    wlnt = Wln.T
    bln2 = bln.reshape(1, H)
    whr, whz, whn = Whh[0:H].T, Whh[H:2 * H].T, Whh[2 * H:3 * H].T
    bhr, bhz, bhn = (bhh[0:H].reshape(1, H), bhh[H:2 * H].reshape(1, H),
                     bhh[2 * H:3 * H].reshape(1, H))
    wir, wiz, win = Wih[0:H].T, Wih[H:2 * H].T, Wih[2 * H:3 * H].T
    bir, biz, bin_ = (bih[0:H].reshape(1, H), bih[H:2 * H].reshape(1, H),
                      bih[2 * H:3 * H].reshape(1, H))

    sc_gather, sc_scatter = _sc_kernels()
    degp = sc_scatter(ones_rows, dst2d, zeros_rows)
    nf, h, ghr, ghz, ghn = _pro_call(
        x, wpt, bp2, wlnt, bln2, whr, whz, whn, bhr, bhz, bhn)
    hidden = nf
    for _ in range(STEPS):
        xj = sc_gather(h, src)
        m = _edge_call(edge_attr, xj, w1t, b1, w2t, b2, brep, ssum)
        p = sc_scatter(m, dst2d, zeros_rows)
        nf, h = _gru_call(p, degp, ghr, ghz, ghn, hidden,
                          wir, wiz, win, bir, biz, bin_, wlnt, bln2)
    return nf
